# Initial kernel scaffold; baseline (speedup 1.0000x reference)
import jax
import jax.numpy as jnp
from jax.experimental import pallas as pl


def kernel(x, edge_index, W, b):
    raise NotImplementedError("write your pallas kernel here")


# speedup vs baseline: 17.3855x; 17.3855x over previous
import functools

import jax
import jax.numpy as jnp
from jax import lax
from jax.experimental import pallas as pl
from jax.experimental.pallas import tpu as pltpu
from jax.experimental.pallas import tpu_sc as plsc

_L = 16
_NC = 2
_NS = 16
_NW = _NC * _NS
_BLK = 512


def _pick_chunk(ew):
  for c in range(128, 7, -8):
    if ew % c == 0:
      return c
  raise ValueError(f"no valid chunk for per-worker edge count {ew}")


def _make_deg(N_pad, E):
  ew = E // _NW
  chunk = _pick_chunk(ew)
  steps = ew // chunk
  rows = N_pad // _NS
  mesh = plsc.VectorSubcoreMesh(core_axis_name="c", subcore_axis_name="s")

  @functools.partial(
      pl.kernel,
      out_type=jax.ShapeDtypeStruct((_NC, N_pad), jnp.float32),
      mesh=mesh,
      scratch_types=[
          pltpu.VMEM((chunk,), jnp.int32),
          pltpu.VMEM((chunk,), jnp.float32),
          pltpu.VMEM((rows,), jnp.float32),
          pltpu.VMEM_SHARED((N_pad,), jnp.float32),
      ],
  )
  def deg_kernel(dst_hbm, deg_hbm, idx_v, ones_v, zero_v, deg_sh):
    c = lax.axis_index("c")
    s = lax.axis_index("s")
    u = c * _NS + s

    one16 = jnp.ones((_L,), jnp.float32)
    zero16 = jnp.zeros((_L,), jnp.float32)

    def fill_ones(i, _):
      ones_v[pl.ds(i * _L, _L)] = one16
      return 0

    lax.fori_loop(0, chunk // _L, fill_ones, 0)

    def fill_zero(i, _):
      zero_v[pl.ds(i * _L, _L)] = zero16
      return 0

    lax.fori_loop(0, rows // _L, fill_zero, 0)

    pltpu.sync_copy(zero_v, deg_sh.at[pl.ds(s * rows, rows)])
    plsc.subcore_barrier()

    def step(i, _):
      base = u * ew + i * chunk
      pltpu.sync_copy(dst_hbm.at[pl.ds(base, chunk)], idx_v)
      pltpu.sync_copy(ones_v, deg_sh.at[idx_v], add=True)
      return 0

    lax.fori_loop(0, steps, step, 0)
    plsc.subcore_barrier()
    pltpu.sync_copy(deg_sh.at[pl.ds(s * rows, rows)],
                    deg_hbm.at[c, pl.ds(s * rows, rows)])

  return deg_kernel


def _make_agg(N_pad, D, E):
  ew = E // _NW
  chunk = _pick_chunk(ew)
  steps = ew // chunk
  rows = N_pad // _NS
  mesh = plsc.VectorSubcoreMesh(core_axis_name="c", subcore_axis_name="s")

  @functools.partial(
      pl.kernel,
      out_type=jax.ShapeDtypeStruct((_NC, N_pad, D), jnp.float32),
      mesh=mesh,
      scratch_types=[
          pltpu.VMEM((chunk,), jnp.int32),
          pltpu.VMEM((chunk,), jnp.int32),
          pltpu.VMEM((chunk, D), jnp.float32),
          pltpu.SemaphoreType.DMA,
          pltpu.VMEM_SHARED((N_pad, D), jnp.float32),
      ],
  )
  def agg_kernel(hp_hbm, zeros_hbm, src_hbm, dst_hbm, out_hbm,
                 src_v, dst_v, gbuf, sem, acc_sh):
    c = lax.axis_index("c")
    s = lax.axis_index("s")
    u = c * _NS + s
    r0 = s * rows

    @pl.when(c == 0)
    def _():
      pltpu.sync_copy(hp_hbm.at[pl.ds(r0, rows)], acc_sh.at[pl.ds(r0, rows)])

    @pl.when(c != 0)
    def _():
      pltpu.sync_copy(zeros_hbm.at[pl.ds(r0, rows)],
                      acc_sh.at[pl.ds(r0, rows)])

    plsc.subcore_barrier()

    def step(i, _):
      base = u * ew + i * chunk
      pltpu.sync_copy(src_hbm.at[pl.ds(base, chunk)], src_v)
      pltpu.sync_copy(dst_hbm.at[pl.ds(base, chunk)], dst_v)
      pltpu.async_copy(hp_hbm.at[src_v], gbuf, sem).wait()
      pltpu.sync_copy(gbuf, acc_sh.at[dst_v], add=True)
      return 0

    lax.fori_loop(0, steps, step, 0)
    plsc.subcore_barrier()
    pltpu.sync_copy(acc_sh.at[pl.ds(r0, rows)],
                    out_hbm.at[c, pl.ds(r0, rows)])

  return agg_kernel


def _scale_body(x_ref, w_ref, d0_ref, d1_ref, hp_ref, dis_ref):
  deg = d0_ref[0, 0] + d1_ref[0, 0] + 1.0
  dis = lax.rsqrt(deg)
  dis_ref[...] = dis
  h = jnp.dot(x_ref[...], w_ref[...], preferred_element_type=jnp.float32)
  hp_ref[...] = h * dis[:, None]


def _finish_body(p0_ref, p1_ref, dis_ref, b_ref, out_ref):
  acc = (p0_ref[0] + p1_ref[0]) * dis_ref[...][:, None] + b_ref[...][None, :]
  m = jnp.max(acc, axis=1, keepdims=True)
  lse = jnp.log(jnp.sum(jnp.exp(acc - m), axis=1, keepdims=True)) + m
  out_ref[...] = acc - lse


def kernel(x, edge_index, W, b):
  N, D_in = x.shape
  D = W.shape[1]
  E = edge_index.shape[1]
  unit = 2560
  N_pad = ((N + unit - 1) // unit) * unit
  grid = N_pad // _BLK

  src = edge_index[0].astype(jnp.int32)
  dst = edge_index[1].astype(jnp.int32)
  x_pad = jnp.pad(x, ((0, N_pad - N), (0, 0)))
  zeros2d = jnp.zeros((N_pad, D), jnp.float32)

  deg_parts = _make_deg(N_pad, E)(dst)
  deg_parts3 = deg_parts.reshape(_NC, 1, N_pad)

  hp, dis = pl.pallas_call(
      _scale_body,
      grid=(grid,),
      in_specs=[
          pl.BlockSpec((_BLK, D_in), lambda i: (i, 0)),
          pl.BlockSpec((D_in, D), lambda i: (0, 0)),
          pl.BlockSpec((1, 1, _BLK), lambda i: (0, 0, i)),
          pl.BlockSpec((1, 1, _BLK), lambda i: (1, 0, i)),
      ],
      out_specs=[
          pl.BlockSpec((_BLK, D), lambda i: (i, 0)),
          pl.BlockSpec((_BLK,), lambda i: (i,)),
      ],
      out_shape=[
          jax.ShapeDtypeStruct((N_pad, D), jnp.float32),
          jax.ShapeDtypeStruct((N_pad,), jnp.float32),
      ],
  )(x_pad, W, deg_parts3, deg_parts3)

  parts = _make_agg(N_pad, D, E)(hp, zeros2d, src, dst)

  out = pl.pallas_call(
      _finish_body,
      grid=(grid,),
      in_specs=[
          pl.BlockSpec((1, _BLK, D), lambda i: (0, i, 0)),
          pl.BlockSpec((1, _BLK, D), lambda i: (1, i, 0)),
          pl.BlockSpec((_BLK,), lambda i: (i,)),
          pl.BlockSpec((D,), lambda i: (0,)),
      ],
      out_specs=pl.BlockSpec((_BLK, D), lambda i: (i, 0)),
      out_shape=jax.ShapeDtypeStruct((N_pad, D), jnp.float32),
  )(parts, parts, dis, b)

  return out[:N]


# speedup vs baseline: 29.2242x; 1.6810x over previous
import functools

import jax
import jax.numpy as jnp
from jax import lax
from jax.experimental import pallas as pl
from jax.experimental.pallas import tpu as pltpu
from jax.experimental.pallas import tpu_sc as plsc

_L = 16
_NC = 2
_NS = 16
_NW = _NC * _NS
_BLK = 512


def _pick_chunk(ew):
  for c in range(128, 7, -8):
    if ew % c == 0:
      return c
  raise ValueError(f"no valid chunk for per-worker edge count {ew}")


def _make_deg(N_pad, E):
  ew = E // _NW
  chunk = _pick_chunk(ew)
  steps = ew // chunk
  rows = N_pad // _NS
  mesh = plsc.VectorSubcoreMesh(core_axis_name="c", subcore_axis_name="s")

  @functools.partial(
      pl.kernel,
      out_type=jax.ShapeDtypeStruct((_NC, N_pad), jnp.float32),
      mesh=mesh,
      scratch_types=[
          pltpu.VMEM((chunk,), jnp.int32),
          pltpu.VMEM((chunk,), jnp.float32),
          pltpu.VMEM((rows,), jnp.float32),
          pltpu.VMEM_SHARED((N_pad,), jnp.float32),
      ],
  )
  def deg_kernel(dst_hbm, deg_hbm, idx_v, ones_v, zero_v, deg_sh):
    c = lax.axis_index("c")
    s = lax.axis_index("s")
    u = c * _NS + s

    one16 = jnp.ones((_L,), jnp.float32)
    zero16 = jnp.zeros((_L,), jnp.float32)

    def fill_ones(i, _):
      ones_v[pl.ds(i * _L, _L)] = one16
      return 0

    lax.fori_loop(0, chunk // _L, fill_ones, 0)

    def fill_zero(i, _):
      zero_v[pl.ds(i * _L, _L)] = zero16
      return 0

    lax.fori_loop(0, rows // _L, fill_zero, 0)

    pltpu.sync_copy(zero_v, deg_sh.at[pl.ds(s * rows, rows)])
    plsc.subcore_barrier()

    def step(i, _):
      base = u * ew + i * chunk
      pltpu.sync_copy(dst_hbm.at[pl.ds(base, chunk)], idx_v)
      pltpu.sync_copy(ones_v, deg_sh.at[idx_v], add=True)
      return 0

    lax.fori_loop(0, steps, step, 0)
    plsc.subcore_barrier()
    pltpu.sync_copy(deg_sh.at[pl.ds(s * rows, rows)],
                    deg_hbm.at[c, pl.ds(s * rows, rows)])

  return deg_kernel


def _pick_block(steps):
  for b in range(32, 0, -1):
    if steps % b == 0:
      return b
  return 1


def _make_agg(N_pad, D, E):
  ew = E // _NW
  chunk = _pick_chunk(ew)
  steps = ew // chunk
  bs = _pick_block(steps)
  nblk = steps // bs
  rows = N_pad // _NS
  mesh = plsc.VectorSubcoreMesh(core_axis_name="c", subcore_axis_name="s")

  @functools.partial(
      pl.kernel,
      out_type=jax.ShapeDtypeStruct((_NC, N_pad, D), jnp.float32),
      mesh=mesh,
      scratch_types=[
          pltpu.VMEM((bs, chunk), jnp.int32),
          pltpu.VMEM((bs, chunk), jnp.int32),
          pltpu.VMEM((chunk, D), jnp.float32),
          pltpu.VMEM((chunk, D), jnp.float32),
          pltpu.SemaphoreType.DMA,
          pltpu.SemaphoreType.DMA,
          pltpu.VMEM_SHARED((N_pad, D), jnp.float32),
      ],
  )
  def agg_kernel(hp_hbm, zeros_hbm, src_hbm, dst_hbm, out_hbm,
                 src_v, dst_v, buf_a, buf_b, sem_a, sem_b, acc_sh):
    c = lax.axis_index("c")
    s = lax.axis_index("s")
    u = c * _NS + s
    r0 = s * rows

    @pl.when(c == 0)
    def _():
      pltpu.sync_copy(hp_hbm.at[pl.ds(r0, rows)], acc_sh.at[pl.ds(r0, rows)])

    @pl.when(c != 0)
    def _():
      pltpu.sync_copy(zeros_hbm.at[pl.ds(r0, rows)],
                      acc_sh.at[pl.ds(r0, rows)])

    plsc.subcore_barrier()

    def block(b, _):
      pltpu.sync_copy(src_hbm.at[u, b], src_v)
      pltpu.sync_copy(dst_hbm.at[u, b], dst_v)
      pltpu.async_copy(hp_hbm.at[src_v.at[0]], buf_a, sem_a)

      def pair(g, _):
        j = 2 * g
        pltpu.async_copy(hp_hbm.at[src_v.at[j + 1]], buf_b, sem_b)
        pltpu.make_async_copy(hp_hbm.at[src_v.at[j]], buf_a, sem_a).wait()
        pltpu.sync_copy(buf_a, acc_sh.at[dst_v.at[j]], add=True)
        pltpu.async_copy(hp_hbm.at[src_v.at[j + 2]], buf_a, sem_a)
        pltpu.make_async_copy(hp_hbm.at[src_v.at[j + 1]], buf_b, sem_b).wait()
        pltpu.sync_copy(buf_b, acc_sh.at[dst_v.at[j + 1]], add=True)
        return 0

      if bs % 2 == 1:
        lax.fori_loop(0, bs // 2, pair, 0)
        last = bs - 1
        pltpu.make_async_copy(hp_hbm.at[src_v.at[last]], buf_a, sem_a).wait()
        pltpu.sync_copy(buf_a, acc_sh.at[dst_v.at[last]], add=True)
      else:
        lax.fori_loop(0, bs // 2 - 1, pair, 0)
        j = bs - 2
        pltpu.async_copy(hp_hbm.at[src_v.at[j + 1]], buf_b, sem_b)
        pltpu.make_async_copy(hp_hbm.at[src_v.at[j]], buf_a, sem_a).wait()
        pltpu.sync_copy(buf_a, acc_sh.at[dst_v.at[j]], add=True)
        pltpu.make_async_copy(hp_hbm.at[src_v.at[j + 1]], buf_b, sem_b).wait()
        pltpu.sync_copy(buf_b, acc_sh.at[dst_v.at[j + 1]], add=True)
      return 0

    lax.fori_loop(0, nblk, block, 0)
    plsc.subcore_barrier()
    pltpu.sync_copy(acc_sh.at[pl.ds(r0, rows)],
                    out_hbm.at[c, pl.ds(r0, rows)])

  return agg_kernel


def _scale_body(x_ref, w_ref, d0_ref, d1_ref, hp_ref, dis_ref):
  deg = d0_ref[0, 0] + d1_ref[0, 0] + 1.0
  dis = lax.rsqrt(deg)
  dis_ref[...] = dis
  h = jnp.dot(x_ref[...], w_ref[...], preferred_element_type=jnp.float32)
  hp_ref[...] = h * dis[:, None]


def _finish_body(p0_ref, p1_ref, dis_ref, b_ref, out_ref):
  acc = (p0_ref[0] + p1_ref[0]) * dis_ref[...][:, None] + b_ref[...][None, :]
  m = jnp.max(acc, axis=1, keepdims=True)
  lse = jnp.log(jnp.sum(jnp.exp(acc - m), axis=1, keepdims=True)) + m
  out_ref[...] = acc - lse


def kernel(x, edge_index, W, b):
  N, D_in = x.shape
  D = W.shape[1]
  E = edge_index.shape[1]
  unit = 2560
  N_pad = ((N + unit - 1) // unit) * unit
  grid = N_pad // _BLK

  src = edge_index[0].astype(jnp.int32)
  dst = edge_index[1].astype(jnp.int32)
  x_pad = jnp.pad(x, ((0, N_pad - N), (0, 0)))
  zeros2d = jnp.zeros((N_pad, D), jnp.float32)

  deg_parts = _make_deg(N_pad, E)(dst)
  deg_parts3 = deg_parts.reshape(_NC, 1, N_pad)

  hp, dis = pl.pallas_call(
      _scale_body,
      grid=(grid,),
      in_specs=[
          pl.BlockSpec((_BLK, D_in), lambda i: (i, 0)),
          pl.BlockSpec((D_in, D), lambda i: (0, 0)),
          pl.BlockSpec((1, 1, _BLK), lambda i: (0, 0, i)),
          pl.BlockSpec((1, 1, _BLK), lambda i: (1, 0, i)),
      ],
      out_specs=[
          pl.BlockSpec((_BLK, D), lambda i: (i, 0)),
          pl.BlockSpec((_BLK,), lambda i: (i,)),
      ],
      out_shape=[
          jax.ShapeDtypeStruct((N_pad, D), jnp.float32),
          jax.ShapeDtypeStruct((N_pad,), jnp.float32),
      ],
  )(x_pad, W, deg_parts3, deg_parts3)

  agg_chunk = _pick_chunk(E // _NW)
  agg_steps = E // _NW // agg_chunk
  agg_bs = _pick_block(agg_steps)
  src4d = src.reshape(_NW, agg_steps // agg_bs, agg_bs, agg_chunk)
  dst4d = dst.reshape(_NW, agg_steps // agg_bs, agg_bs, agg_chunk)
  parts = _make_agg(N_pad, D, E)(hp, zeros2d, src4d, dst4d)

  out = pl.pallas_call(
      _finish_body,
      grid=(grid,),
      in_specs=[
          pl.BlockSpec((1, _BLK, D), lambda i: (0, i, 0)),
          pl.BlockSpec((1, _BLK, D), lambda i: (1, i, 0)),
          pl.BlockSpec((_BLK,), lambda i: (i,)),
          pl.BlockSpec((D,), lambda i: (0,)),
      ],
      out_specs=pl.BlockSpec((_BLK, D), lambda i: (i, 0)),
      out_shape=jax.ShapeDtypeStruct((N_pad, D), jnp.float32),
  )(parts, parts, dis, b)

  return out[:N]


# speedup vs baseline: 37.2340x; 1.2741x over previous
import functools

import jax
import jax.numpy as jnp
from jax import lax
from jax.experimental import pallas as pl
from jax.experimental.pallas import tpu as pltpu
from jax.experimental.pallas import tpu_sc as plsc

_L = 16
_NC = 2
_NS = 16
_NW = _NC * _NS
_BLK = 512


def _pick_chunk(ew):
  for c in range(128, 7, -8):
    if ew % c == 0:
      return c
  raise ValueError(f"no valid chunk for per-worker edge count {ew}")


def _make_deg(N_pad, E):
  ew = E // _NW
  nvec = ew // _L
  rows = N_pad // _NS
  mesh = plsc.VectorSubcoreMesh(core_axis_name="c", subcore_axis_name="s")

  @functools.partial(
      pl.kernel,
      out_type=jax.ShapeDtypeStruct((_NW, 1, N_pad), jnp.float32),
      mesh=mesh,
      scratch_types=[
          pltpu.VMEM((ew,), jnp.int32),
          pltpu.VMEM((N_pad,), jnp.float32),
      ],
      compiler_params=pltpu.CompilerParams(needs_layout_passes=False),
  )
  def deg_kernel(dst_hbm, deg_hbm, idx_v, hist_v):
    c = lax.axis_index("c")
    s = lax.axis_index("s")
    u = c * _NS + s

    one16 = jnp.ones((_L,), jnp.float32)
    zero16 = jnp.zeros((_L,), jnp.float32)

    pltpu.sync_copy(dst_hbm.at[u, 0], idx_v)

    def fill_zero(i, _):
      hist_v[pl.ds(i * _L, _L)] = zero16
      return 0

    lax.fori_loop(0, N_pad // _L, fill_zero, 0)

    def step(j, _):
      idx16 = idx_v[pl.ds(j * _L, _L)]
      plsc.addupdate_scatter(hist_v, [idx16], one16)
      return 0

    lax.fori_loop(0, nvec, step, 0)

    pltpu.sync_copy(hist_v, deg_hbm.at[u, 0])

  return deg_kernel


def _pick_block(steps):
  for b in range(32, 0, -1):
    if steps % b == 0:
      return b
  return 1


def _make_agg(N_pad, D, E):
  ew = E // _NW
  chunk = _pick_chunk(ew)
  steps = ew // chunk
  bs = _pick_block(steps)
  nblk = steps // bs
  rows = N_pad // _NS
  mesh = plsc.VectorSubcoreMesh(core_axis_name="c", subcore_axis_name="s")

  @functools.partial(
      pl.kernel,
      out_type=jax.ShapeDtypeStruct((_NC, N_pad, D), jnp.float32),
      mesh=mesh,
      scratch_types=[
          pltpu.VMEM((bs, chunk), jnp.int32),
          pltpu.VMEM((bs, chunk), jnp.int32),
          pltpu.VMEM((chunk, D), jnp.float32),
          pltpu.VMEM((chunk, D), jnp.float32),
          pltpu.SemaphoreType.DMA,
          pltpu.SemaphoreType.DMA,
          pltpu.VMEM_SHARED((N_pad, D), jnp.float32),
      ],
  )
  def agg_kernel(hp_hbm, zeros_hbm, src_hbm, dst_hbm, out_hbm,
                 src_v, dst_v, buf_a, buf_b, sem_a, sem_b, acc_sh):
    c = lax.axis_index("c")
    s = lax.axis_index("s")
    u = c * _NS + s
    r0 = s * rows

    @pl.when(c == 0)
    def _():
      pltpu.sync_copy(hp_hbm.at[pl.ds(r0, rows)], acc_sh.at[pl.ds(r0, rows)])

    @pl.when(c != 0)
    def _():
      pltpu.sync_copy(zeros_hbm.at[pl.ds(r0, rows)],
                      acc_sh.at[pl.ds(r0, rows)])

    plsc.subcore_barrier()

    def block(b, _):
      pltpu.sync_copy(src_hbm.at[u, b], src_v)
      pltpu.sync_copy(dst_hbm.at[u, b], dst_v)
      pltpu.async_copy(hp_hbm.at[src_v.at[0]], buf_a, sem_a)

      def pair(g, _):
        j = 2 * g
        pltpu.async_copy(hp_hbm.at[src_v.at[j + 1]], buf_b, sem_b)
        pltpu.make_async_copy(hp_hbm.at[src_v.at[j]], buf_a, sem_a).wait()
        pltpu.sync_copy(buf_a, acc_sh.at[dst_v.at[j]], add=True)
        pltpu.async_copy(hp_hbm.at[src_v.at[j + 2]], buf_a, sem_a)
        pltpu.make_async_copy(hp_hbm.at[src_v.at[j + 1]], buf_b, sem_b).wait()
        pltpu.sync_copy(buf_b, acc_sh.at[dst_v.at[j + 1]], add=True)
        return 0

      if bs % 2 == 1:
        lax.fori_loop(0, bs // 2, pair, 0)
        last = bs - 1
        pltpu.make_async_copy(hp_hbm.at[src_v.at[last]], buf_a, sem_a).wait()
        pltpu.sync_copy(buf_a, acc_sh.at[dst_v.at[last]], add=True)
      else:
        lax.fori_loop(0, bs // 2 - 1, pair, 0)
        j = bs - 2
        pltpu.async_copy(hp_hbm.at[src_v.at[j + 1]], buf_b, sem_b)
        pltpu.make_async_copy(hp_hbm.at[src_v.at[j]], buf_a, sem_a).wait()
        pltpu.sync_copy(buf_a, acc_sh.at[dst_v.at[j]], add=True)
        pltpu.make_async_copy(hp_hbm.at[src_v.at[j + 1]], buf_b, sem_b).wait()
        pltpu.sync_copy(buf_b, acc_sh.at[dst_v.at[j + 1]], add=True)
      return 0

    lax.fori_loop(0, nblk, block, 0)
    plsc.subcore_barrier()
    pltpu.sync_copy(acc_sh.at[pl.ds(r0, rows)],
                    out_hbm.at[c, pl.ds(r0, rows)])

  return agg_kernel


def _scale_body(x_ref, w_ref, d_ref, hp_ref, dis_ref):
  deg = jnp.sum(d_ref[:, 0, :], axis=0) + 1.0
  dis = lax.rsqrt(deg)
  dis_ref[...] = dis
  h = jnp.dot(x_ref[...], w_ref[...], preferred_element_type=jnp.float32)
  hp_ref[...] = h * dis[:, None]


def _finish_body(p0_ref, p1_ref, dis_ref, b_ref, out_ref):
  acc = (p0_ref[0] + p1_ref[0]) * dis_ref[...][:, None] + b_ref[...][None, :]
  m = jnp.max(acc, axis=1, keepdims=True)
  lse = jnp.log(jnp.sum(jnp.exp(acc - m), axis=1, keepdims=True)) + m
  out_ref[...] = acc - lse


def kernel(x, edge_index, W, b):
  N, D_in = x.shape
  D = W.shape[1]
  E = edge_index.shape[1]
  unit = 2560
  N_pad = ((N + unit - 1) // unit) * unit
  grid = N_pad // _BLK

  src = edge_index[0].astype(jnp.int32)
  dst = edge_index[1].astype(jnp.int32)
  x_pad = jnp.pad(x, ((0, N_pad - N), (0, 0)))
  zeros2d = jnp.zeros((N_pad, D), jnp.float32)

  dst3d_deg = dst.reshape(_NW, 1, E // _NW)
  deg_parts3 = _make_deg(N_pad, E)(dst3d_deg)

  hp, dis = pl.pallas_call(
      _scale_body,
      grid=(grid,),
      in_specs=[
          pl.BlockSpec((_BLK, D_in), lambda i: (i, 0)),
          pl.BlockSpec((D_in, D), lambda i: (0, 0)),
          pl.BlockSpec((_NW, 1, _BLK), lambda i: (0, 0, i)),
      ],
      out_specs=[
          pl.BlockSpec((_BLK, D), lambda i: (i, 0)),
          pl.BlockSpec((_BLK,), lambda i: (i,)),
      ],
      out_shape=[
          jax.ShapeDtypeStruct((N_pad, D), jnp.float32),
          jax.ShapeDtypeStruct((N_pad,), jnp.float32),
      ],
  )(x_pad, W, deg_parts3)

  agg_chunk = _pick_chunk(E // _NW)
  agg_steps = E // _NW // agg_chunk
  agg_bs = _pick_block(agg_steps)
  src4d = src.reshape(_NW, agg_steps // agg_bs, agg_bs, agg_chunk)
  dst4d = dst.reshape(_NW, agg_steps // agg_bs, agg_bs, agg_chunk)
  parts = _make_agg(N_pad, D, E)(hp, zeros2d, src4d, dst4d)

  out = pl.pallas_call(
      _finish_body,
      grid=(grid,),
      in_specs=[
          pl.BlockSpec((1, _BLK, D), lambda i: (0, i, 0)),
          pl.BlockSpec((1, _BLK, D), lambda i: (1, i, 0)),
          pl.BlockSpec((_BLK,), lambda i: (i,)),
          pl.BlockSpec((D,), lambda i: (0,)),
      ],
      out_specs=pl.BlockSpec((_BLK, D), lambda i: (i, 0)),
      out_shape=jax.ShapeDtypeStruct((N_pad, D), jnp.float32),
  )(parts, parts, dis, b)

  return out[:N]
